# baseline (speedup 1.0000x reference)
import functools

import jax
import jax.numpy as jnp
from jax import lax
from jax.experimental import pallas as pl
from jax.experimental.pallas import tpu as pltpu
from jax.experimental.pallas import tpu_sc as plsc

B, L, INTD, HD = 4096, 200, 64, 192
N = B * L
HDP = 256

NC, NS = 2, 16
NW = NC * NS
CHUNK = 64
LANES = 16


def _proj(E, Wk, bias):
    n, d = E.shape
    bm = min(n, 512)
    grid = (pl.cdiv(n, bm),)

    def body(e_ref, w_ref, b_ref, o_ref):
        o_ref[...] = (
            jnp.dot(e_ref[...], w_ref[...], preferred_element_type=jnp.float32)
            + b_ref[...]
        )

    return pl.pallas_call(
        body,
        grid=grid,
        in_specs=[
            pl.BlockSpec((bm, d), lambda i: (i, 0)),
            pl.BlockSpec((d, HDP), lambda i: (0, 0)),
            pl.BlockSpec((1, HDP), lambda i: (0, 0)),
        ],
        out_specs=pl.BlockSpec((bm, HDP), lambda i: (i, 0)),
        out_shape=jax.ShapeDtypeStruct((n, HDP), jnp.float32),
    )(E, Wk, bias)


def _make_gather_sum():
    per_w = N // NW
    chunks = per_w // CHUNK
    mesh = plsc.VectorSubcoreMesh(core_axis_name="c", subcore_axis_name="s")

    @functools.partial(
        pl.kernel,
        out_type=jax.ShapeDtypeStruct((N, HD), jnp.float32),
        mesh=mesh,
        scratch_types=[
            pltpu.VMEM((CHUNK,), jnp.int32),
            pltpu.VMEM((CHUNK,), jnp.int32),
            pltpu.VMEM((CHUNK,), jnp.int32),
            pltpu.VMEM((CHUNK,), jnp.int32),
            pltpu.VMEM((CHUNK,), jnp.int32),
            pltpu.VMEM((CHUNK, HDP), jnp.float32),
            pltpu.VMEM((CHUNK, HDP), jnp.float32),
            pltpu.VMEM((CHUNK, HDP), jnp.float32),
            pltpu.VMEM((CHUNK, HDP), jnp.float32),
            pltpu.VMEM((CHUNK, HDP), jnp.float32),
            pltpu.VMEM((CHUNK, HD), jnp.float32),
            pltpu.SemaphoreType.DMA,
        ],
    )
    def gather_sum(t0, t1, t2, t3, t4, i0, i1, i2, i3, i4, out,
                   v0, v1, v2, v3, v4, b0, b1, b2, b3, b4, ov, sem):
        wid = lax.axis_index("s") * NC + lax.axis_index("c")
        base0 = wid * per_w
        tables = (t0, t1, t2, t3, t4)
        idxs = (i0, i1, i2, i3, i4)
        idxv = (v0, v1, v2, v3, v4)
        bufs = (b0, b1, b2, b3, b4)

        def chunk_body(t, carry):
            base = base0 + t * CHUNK
            for k in range(5):
                pltpu.sync_copy(idxs[k].at[pl.ds(base, CHUNK)], idxv[k])
            cps = [
                pltpu.async_copy(tables[k].at[idxv[k]], bufs[k], sem)
                for k in range(5)
            ]
            for cp in cps:
                cp.wait()

            def sum_body(c, carry2):
                for d in range(HD // LANES):
                    sl = pl.ds(d * LANES, LANES)
                    s = (bufs[0][c, sl] + bufs[1][c, sl] + bufs[2][c, sl]
                         + bufs[3][c, sl] + bufs[4][c, sl])
                    ov[c, sl] = s
                return carry2

            lax.fori_loop(0, CHUNK, sum_body, 0, unroll=False)
            pltpu.sync_copy(ov, out.at[pl.ds(base, CHUNK)])
            return carry

        lax.fori_loop(0, chunks, chunk_body, 0, unroll=False)

    return gather_sum


_gather_sum = _make_gather_sum()


def kernel(test, question, tag, correct, elapsed_question, mask, interaction,
           extra, E_int, E_test, E_q, E_tag, E_el, W, b):
    pad = ((0, 0), (0, HDP - HD))
    zero = jnp.zeros((1, HDP), jnp.float32)
    bias = jnp.pad(b.reshape(1, HD), pad)
    Wp = [jnp.pad(W[k * INTD:(k + 1) * INTD], pad) for k in range(5)]
    P_int = _proj(E_int, Wp[0], bias)
    P_test = _proj(E_test, Wp[1], zero)
    P_q = _proj(E_q, Wp[2], zero)
    P_tag = _proj(E_tag, Wp[3], zero)
    P_el = _proj(E_el, Wp[4], zero)

    i_int = interaction.reshape(N).astype(jnp.int32)
    i_test = test.reshape(N).astype(jnp.int32)
    i_q = question.reshape(N).astype(jnp.int32)
    i_tag = tag.reshape(N).astype(jnp.int32)
    i_el = elapsed_question.reshape(N).astype(jnp.int32)

    out = _gather_sum(P_int, P_test, P_q, P_tag, P_el,
                      i_int, i_test, i_q, i_tag, i_el)
    return out.reshape(B, L, HD)


# speedup vs baseline: 5.0488x; 5.0488x over previous
import functools

import jax
import jax.numpy as jnp
from jax import lax
from jax.experimental import pallas as pl
from jax.experimental.pallas import tpu as pltpu
from jax.experimental.pallas import tpu_sc as plsc

B, L, INTD, HD = 4096, 200, 64, 192
N = B * L
HDP = 256

NC, NS = 2, 16
NW = NC * NS
CHUNK = 64
G = 8
LANES = 16

N_TEST, N_Q, N_TAG, N_EL, N_INT = 1001, 100001, 1001, 301, 3


def _proj(E, Wk, bias):
    n, d = E.shape
    bm = min(n, 512)
    grid = (pl.cdiv(n, bm),)

    def body(e_ref, w_ref, b_ref, o_ref):
        o_ref[...] = (
            jnp.dot(e_ref[...], w_ref[...], preferred_element_type=jnp.float32)
            + b_ref[...]
        )

    return pl.pallas_call(
        body,
        grid=grid,
        in_specs=[
            pl.BlockSpec((bm, d), lambda i: (i, 0)),
            pl.BlockSpec((d, HDP), lambda i: (0, 0)),
            pl.BlockSpec((1, HDP), lambda i: (0, 0)),
        ],
        out_specs=pl.BlockSpec((bm, HDP), lambda i: (i, 0)),
        out_shape=jax.ShapeDtypeStruct((n, HDP), jnp.float32),
    )(E, Wk, bias)


def _pair_table(P_big, P_one, n_big, n_one):
    def body(pb_ref, po_ref, o_ref):
        o_ref[...] = pb_ref[...] + po_ref[...]

    return pl.pallas_call(
        body,
        grid=(n_one,),
        in_specs=[
            pl.BlockSpec((1, n_big, HDP), lambda i: (0, 0, 0)),
            pl.BlockSpec((1, 1, HDP), lambda i: (i, 0, 0)),
        ],
        out_specs=pl.BlockSpec((1, n_big, HDP), lambda i: (i, 0, 0)),
        out_shape=jax.ShapeDtypeStruct((n_one, n_big, HDP), jnp.float32),
    )(P_big.reshape(1, n_big, HDP), P_one.reshape(n_one, 1, HDP))


def _make_gather_sum():
    per_w = N // NW
    sup_len = G * CHUNK
    sups = per_w // sup_len
    mesh = plsc.VectorSubcoreMesh(core_axis_name="c", subcore_axis_name="s")

    @functools.partial(
        pl.kernel,
        out_type=jax.ShapeDtypeStruct((N, HD), jnp.float32),
        mesh=mesh,
        scratch_types=[
            pltpu.VMEM((G, CHUNK), jnp.int32),
            pltpu.VMEM((G, CHUNK), jnp.int32),
            pltpu.VMEM((G, CHUNK), jnp.int32),
            pltpu.VMEM((G, CHUNK), jnp.int32),
            pltpu.VMEM((G, CHUNK), jnp.int32),
            pltpu.VMEM((2, CHUNK, HDP), jnp.float32),
            pltpu.VMEM((2, CHUNK, HDP), jnp.float32),
            pltpu.VMEM((2, CHUNK, HDP), jnp.float32),
            pltpu.VMEM((CHUNK, HD), jnp.float32),
            pltpu.SemaphoreType.DMA,
            pltpu.SemaphoreType.DMA,
            pltpu.SemaphoreType.DMA,
        ],
    )
    def gather_sum(tA, tB, tQ, iInt, iTest, iTag, iEl, iQ, out,
                   rInt, rTest, rTag, rEl, rQ,
                   gA, gB, gQ, ov, sg0, sg1, so):
        wid = lax.axis_index("s") * NC + lax.axis_index("c")
        base0 = wid * per_w
        sgs = (sg0, sg1)

        def issue(j, slot):
            pltpu.async_copy(tA.at[rInt.at[j]], gA.at[slot], sgs[slot])
            pltpu.async_copy(tB.at[rTag.at[j]], gB.at[slot], sgs[slot])
            pltpu.async_copy(tQ.at[rQ.at[j]], gQ.at[slot], sgs[slot])

        def wait_gather(j, slot):
            pltpu.make_async_copy(tA.at[rInt.at[j]], gA.at[slot], sgs[slot]).wait()
            pltpu.make_async_copy(tB.at[rTag.at[j]], gB.at[slot], sgs[slot]).wait()
            pltpu.make_async_copy(tQ.at[rQ.at[j]], gQ.at[slot], sgs[slot]).wait()

        def out_slice(chunk_id):
            return out.at[pl.ds(base0 + chunk_id * CHUNK, CHUNK)]

        def sup_body(s, carry):
            srow = pl.multiple_of((base0 // CHUNK) + s * G, 8)
            sl_idx = pl.ds(srow, G)
            pltpu.sync_copy(iInt.at[sl_idx], rInt)
            pltpu.sync_copy(iTest.at[sl_idx], rTest)
            pltpu.sync_copy(iTag.at[sl_idx], rTag)
            pltpu.sync_copy(iEl.at[sl_idx], rEl)
            pltpu.sync_copy(iQ.at[sl_idx], rQ)

            def comb_body(g, carry2):
                for v in range(CHUNK // LANES):
                    sl = pl.ds(v * LANES, LANES)
                    rInt[g, sl] = rInt[g, sl] * N_TEST + rTest[g, sl]
                    rTag[g, sl] = rTag[g, sl] * N_EL + rEl[g, sl]
                return carry2

            lax.fori_loop(0, G, comb_body, 0, unroll=False)

            issue(0, 0)

            def pair_body(m, carry2):
                for bslot in range(2):
                    j = 2 * m + bslot
                    jg = s * G + j
                    if bslot == 0:
                        issue(j + 1, 1)
                    else:

                        @pl.when(j + 1 < G)
                        def _():
                            issue(j + 1, 0)

                    wait_gather(j, bslot)

                    @pl.when(jg >= 1)
                    def _():
                        prev = jnp.maximum(jg - 1, 0)
                        pltpu.make_async_copy(ov, out_slice(prev), so).wait()

                    def sum_body(c, carry3):
                        for d in range(HD // LANES):
                            sl = pl.ds(d * LANES, LANES)
                            ov[c, sl] = (
                                gA[bslot, c, sl] + gB[bslot, c, sl]
                                + gQ[bslot, c, sl]
                            )
                        return carry3

                    lax.fori_loop(0, CHUNK, sum_body, 0, unroll=False)
                    pltpu.async_copy(ov, out_slice(jg), so)
                return carry2

            lax.fori_loop(0, G // 2, pair_body, 0, unroll=False)
            return carry

        lax.fori_loop(0, sups, sup_body, 0, unroll=False)

        pltpu.make_async_copy(ov, out_slice(sups * G - 1), so).wait()

    return gather_sum


_gather_sum = _make_gather_sum()


def kernel(test, question, tag, correct, elapsed_question, mask, interaction,
           extra, E_int, E_test, E_q, E_tag, E_el, W, b):
    pad = ((0, 0), (0, HDP - HD))
    zero = jnp.zeros((1, HDP), jnp.float32)
    bias = jnp.pad(b.reshape(1, HD), pad)
    Wp = [jnp.pad(W[k * INTD:(k + 1) * INTD], pad) for k in range(5)]

    P_int = _proj(E_int, Wp[0], bias)
    P_test = _proj(E_test, Wp[1], zero)
    P_tag = _proj(E_tag, Wp[3], zero)
    P_el = _proj(E_el, Wp[4], zero)
    tab_A = _pair_table(P_test, P_int, N_TEST, N_INT).reshape(N_INT * N_TEST, HDP)
    tab_B = _pair_table(P_el, P_tag, N_EL, N_TAG).reshape(N_TAG * N_EL, HDP)
    tab_Q = _proj(E_q, Wp[2], zero)

    i_int = interaction.reshape(N // CHUNK, CHUNK).astype(jnp.int32)
    i_test = test.reshape(N // CHUNK, CHUNK).astype(jnp.int32)
    i_q = question.reshape(N // CHUNK, CHUNK).astype(jnp.int32)
    i_tag = tag.reshape(N // CHUNK, CHUNK).astype(jnp.int32)
    i_el = elapsed_question.reshape(N // CHUNK, CHUNK).astype(jnp.int32)

    out = _gather_sum(tab_A, tab_B, tab_Q,
                      i_int, i_test, i_tag, i_el, i_q)
    return out.reshape(B, L, HD)


# speedup vs baseline: 5.4296x; 1.0754x over previous
import functools

import jax
import jax.numpy as jnp
from jax import lax
from jax.experimental import pallas as pl
from jax.experimental.pallas import tpu as pltpu
from jax.experimental.pallas import tpu_sc as plsc

B, L, INTD, HD = 4096, 200, 64, 192
N = B * L
HDP = 256

NC, NS = 2, 16
NW = NC * NS
CHUNK = 64
G = 8
LANES = 16

N_TEST, N_Q, N_TAG, N_EL, N_INT = 1001, 100001, 1001, 301, 3
ST_A = 1008
ST_B = 304


def _proj(E, Wk, bias):
    n, d = E.shape
    bm = min(n, 512)
    grid = (pl.cdiv(n, bm),)

    def body(e_ref, w_ref, b_ref, o_ref):
        o_ref[...] = (
            jnp.dot(e_ref[...], w_ref[...], preferred_element_type=jnp.float32)
            + b_ref[...]
        )

    return pl.pallas_call(
        body,
        grid=grid,
        in_specs=[
            pl.BlockSpec((bm, d), lambda i: (i, 0)),
            pl.BlockSpec((d, HDP), lambda i: (0, 0)),
            pl.BlockSpec((1, HDP), lambda i: (0, 0)),
        ],
        out_specs=pl.BlockSpec((bm, HDP), lambda i: (i, 0)),
        out_shape=jax.ShapeDtypeStruct((n, HDP), jnp.float32),
    )(E, Wk, bias)


def _pair_table(P_big, P_one, stride, n_one):
    def body(pb_ref, po_ref, o_ref):
        o_ref[...] = pb_ref[...] + po_ref[0]

    return pl.pallas_call(
        body,
        grid=(n_one,),
        in_specs=[
            pl.BlockSpec((stride, HDP), lambda i: (0, 0)),
            pl.BlockSpec((1, 1, HDP), lambda i: (i, 0, 0)),
        ],
        out_specs=pl.BlockSpec((stride, HDP), lambda i: (i, 0)),
        out_shape=jax.ShapeDtypeStruct((n_one * stride, HDP), jnp.float32),
    )(P_big, P_one.reshape(n_one, 1, HDP))


def _make_gather_sum():
    per_w = N // NW
    sup_len = G * CHUNK
    sups = per_w // sup_len
    mesh = plsc.VectorSubcoreMesh(core_axis_name="c", subcore_axis_name="s")

    @functools.partial(
        pl.kernel,
        out_type=jax.ShapeDtypeStruct((N, HD), jnp.float32),
        mesh=mesh,
        scratch_types=[
            pltpu.VMEM((G, CHUNK), jnp.int32),
            pltpu.VMEM((G, CHUNK), jnp.int32),
            pltpu.VMEM((G, CHUNK), jnp.int32),
            pltpu.VMEM((G, CHUNK), jnp.int32),
            pltpu.VMEM((G, CHUNK), jnp.int32),
            pltpu.VMEM((2, CHUNK, HDP), jnp.float32),
            pltpu.VMEM((2, CHUNK, HDP), jnp.float32),
            pltpu.VMEM((2, CHUNK, HDP), jnp.float32),
            pltpu.VMEM((CHUNK, HD), jnp.float32),
            pltpu.SemaphoreType.DMA,
            pltpu.SemaphoreType.DMA,
            pltpu.SemaphoreType.DMA,
        ],
    )
    def gather_sum(tA, tB, tQ, iInt, iTest, iTag, iEl, iQ, out,
                   rInt, rTest, rTag, rEl, rQ,
                   gA, gB, gQ, ov, sg0, sg1, so):
        wid = lax.axis_index("s") * NC + lax.axis_index("c")
        base0 = wid * per_w
        sgs = (sg0, sg1)

        def issue(j, slot):
            pltpu.async_copy(tA.at[rInt.at[j]], gA.at[slot], sgs[slot])
            pltpu.async_copy(tB.at[rTag.at[j]], gB.at[slot], sgs[slot])
            pltpu.async_copy(tQ.at[rQ.at[j]], gQ.at[slot], sgs[slot])

        def wait_gather(j, slot):
            pltpu.make_async_copy(tA.at[rInt.at[j]], gA.at[slot], sgs[slot]).wait()
            pltpu.make_async_copy(tB.at[rTag.at[j]], gB.at[slot], sgs[slot]).wait()
            pltpu.make_async_copy(tQ.at[rQ.at[j]], gQ.at[slot], sgs[slot]).wait()

        def out_slice(chunk_id):
            return out.at[pl.ds(base0 + chunk_id * CHUNK, CHUNK)]

        def sup_body(s, carry):
            srow = pl.multiple_of((base0 // CHUNK) + s * G, 8)
            sl_idx = pl.ds(srow, G)
            pltpu.sync_copy(iInt.at[sl_idx], rInt)
            pltpu.sync_copy(iTest.at[sl_idx], rTest)
            pltpu.sync_copy(iTag.at[sl_idx], rTag)
            pltpu.sync_copy(iEl.at[sl_idx], rEl)
            pltpu.sync_copy(iQ.at[sl_idx], rQ)

            def comb_body(g, carry2):
                for v in range(CHUNK // LANES):
                    sl = pl.ds(v * LANES, LANES)
                    rInt[g, sl] = rInt[g, sl] * ST_A + rTest[g, sl]
                    rTag[g, sl] = rTag[g, sl] * ST_B + rEl[g, sl]
                return carry2

            lax.fori_loop(0, G, comb_body, 0, unroll=False)

            issue(0, 0)

            def pair_body(m, carry2):
                for bslot in range(2):
                    j = 2 * m + bslot
                    jg = s * G + j
                    if bslot == 0:
                        issue(j + 1, 1)
                    else:

                        @pl.when(j + 1 < G)
                        def _():
                            issue(j + 1, 0)

                    wait_gather(j, bslot)

                    @pl.when(jg >= 1)
                    def _():
                        prev = jnp.maximum(jg - 1, 0)
                        pltpu.make_async_copy(ov, out_slice(prev), so).wait()

                    def sum_body(c, carry3):
                        for d in range(HD // LANES):
                            sl = pl.ds(d * LANES, LANES)
                            ov[c, sl] = (
                                gA[bslot, c, sl] + gB[bslot, c, sl]
                                + gQ[bslot, c, sl]
                            )
                        return carry3

                    lax.fori_loop(0, CHUNK, sum_body, 0, unroll=False)
                    pltpu.async_copy(ov, out_slice(jg), so)
                return carry2

            lax.fori_loop(0, G // 2, pair_body, 0, unroll=False)
            return carry

        lax.fori_loop(0, sups, sup_body, 0, unroll=False)

        pltpu.make_async_copy(ov, out_slice(sups * G - 1), so).wait()

    return gather_sum


_gather_sum = _make_gather_sum()


def kernel(test, question, tag, correct, elapsed_question, mask, interaction,
           extra, E_int, E_test, E_q, E_tag, E_el, W, b):
    pad = ((0, 0), (0, HDP - HD))
    zero = jnp.zeros((1, HDP), jnp.float32)
    bias = jnp.pad(b.reshape(1, HD), pad)
    Wp = [jnp.pad(W[k * INTD:(k + 1) * INTD], pad) for k in range(5)]

    P_int = _proj(E_int, Wp[0], bias)
    P_test = _proj(E_test, Wp[1], zero)
    P_tag = _proj(E_tag, Wp[3], zero)
    P_el = _proj(E_el, Wp[4], zero)
    P_test_p = jnp.pad(P_test, ((0, ST_A - N_TEST), (0, 0)))
    P_el_p = jnp.pad(P_el, ((0, ST_B - N_EL), (0, 0)))
    tab_A = _pair_table(P_test_p, P_int, ST_A, N_INT)
    tab_B = _pair_table(P_el_p, P_tag, ST_B, N_TAG)
    tab_Q = _proj(E_q, Wp[2], zero)

    i_int = interaction.reshape(N // CHUNK, CHUNK).astype(jnp.int32)
    i_test = test.reshape(N // CHUNK, CHUNK).astype(jnp.int32)
    i_q = question.reshape(N // CHUNK, CHUNK).astype(jnp.int32)
    i_tag = tag.reshape(N // CHUNK, CHUNK).astype(jnp.int32)
    i_el = elapsed_question.reshape(N // CHUNK, CHUNK).astype(jnp.int32)

    out = _gather_sum(tab_A, tab_B, tab_Q,
                      i_int, i_test, i_tag, i_el, i_q)
    return out.reshape(B, L, HD)
